# Initial kernel scaffold; baseline (speedup 1.0000x reference)
import jax
import jax.numpy as jnp
from jax.experimental import pallas as pl


def kernel(x, table, W, b):
    raise NotImplementedError("write your pallas kernel here")


# speedup vs baseline: 9.5840x; 9.5840x over previous
import jax
import jax.numpy as jnp
from jax import lax
from jax.experimental import pallas as pl
from jax.experimental.pallas import tpu as pltpu
from jax.experimental.pallas import tpu_sc as plsc

B = 16384
L = 200
D = 16
NW = 32
ROWS_PER_W = B // NW
CHUNK = 16
NCHUNK = ROWS_PER_W // CHUNK
IDX_PER_CHUNK = CHUNK * L
GATHER_W = 128
NGATHER = IDX_PER_CHUNK // GATHER_W
NCHUNK_G = B * L // IDX_PER_CHUNK
IDX_PAD = 32


def _body(x3, table, wv, bv, out, idx_v, val_v, w_v, b_v, o16_v, gs0, gs1):
    wid = lax.axis_index("s") * 2 + lax.axis_index("c")

    pltpu.sync_copy(wv, w_v)
    pltpu.sync_copy(bv, b_v)

    lane = lax.iota(jnp.int32, D)

    def fire(c, p, sem):
        pltpu.sync_copy(x3.at[c], idx_v.at[p])

        def go(j, _):
            pltpu.async_copy(
                table.at[idx_v.at[p].at[j]],
                val_v.at[p].at[pl.ds(j * GATHER_W, GATHER_W)],
                sem,
            )
            return 0

        lax.fori_loop(0, NGATHER, go, 0)

    def drain(p, sem):
        def go(j, _):
            pltpu.make_async_copy(
                table.at[idx_v.at[p].at[j]],
                val_v.at[p].at[pl.ds(j * GATHER_W, GATHER_W)],
                sem,
            ).wait()
            return 0

        lax.fori_loop(0, NGATHER, go, 0)

    def compute(c, p):
        w_reg = w_v[...]
        vp = val_v.at[p]

        def elem_body(e, sel):
            base = e * L

            def red(l, accs):
                a0, a1, a2, a3 = accs
                r = base + l * 8
                a0 = a0 + vp[r, :]
                a1 = a1 + vp[r + 1, :]
                a2 = a2 + vp[r + 2, :]
                a3 = a3 + vp[r + 3, :]
                a0 = a0 + vp[r + 4, :]
                a1 = a1 + vp[r + 5, :]
                a2 = a2 + vp[r + 6, :]
                a3 = a3 + vp[r + 7, :]
                return a0, a1, a2, a3

            z = jnp.zeros((D,), jnp.float32)
            a0, a1, a2, a3 = lax.fori_loop(0, L // 8, red, (z, z, z, z))
            acc = (a0 + a1) + (a2 + a3)
            v = acc * w_reg
            dnums = lax.GatherDimensionNumbers(
                offset_dims=(), collapsed_slice_dims=(0,), start_index_map=(0,)
            )
            for sh in (8, 4, 2, 1):
                v = v + lax.gather(
                    v,
                    (lane ^ sh)[:, None],
                    dnums,
                    (1,),
                    mode=lax.GatherScatterMode.PROMISE_IN_BOUNDS,
                )
            return jnp.where(lane == e, v, sel)

        sel = lax.fori_loop(0, CHUNK, elem_body, jnp.zeros((D,), jnp.float32))
        zval = sel + b_v[...]
        sig = 1.0 / (1.0 + jnp.exp(-zval))
        r = (sig * 10000.0 + 0.5).astype(jnp.int32).astype(jnp.float32) * 1e-4
        o16_v[...] = r
        pltpu.sync_copy(o16_v, out.at[pl.ds(c * CHUNK, CHUNK)])

    c0 = wid * NCHUNK
    fire(c0, 0, gs0)

    def pair(k, _):
        ca = c0 + 2 * k
        fire(ca + 1, 1, gs1)
        drain(0, gs0)
        compute(ca, 0)

        @pl.when(k < NCHUNK // 2 - 1)
        def _prefetch():
            fire(ca + 2, 0, gs0)

        drain(1, gs1)
        compute(ca + 1, 1)
        return 0

    lax.fori_loop(0, NCHUNK // 2, pair, 0)


@jax.jit
def _run(x3, table, wv, bv):
    mesh = plsc.VectorSubcoreMesh(core_axis_name="c", subcore_axis_name="s")
    return pl.kernel(
        _body,
        out_type=jax.ShapeDtypeStruct((B,), jnp.float32),
        mesh=mesh,
        scratch_types=[
            pltpu.VMEM((2, IDX_PAD, GATHER_W), jnp.int32),
            pltpu.VMEM((2, IDX_PER_CHUNK, D), jnp.float32),
            pltpu.VMEM((D,), jnp.float32),
            pltpu.VMEM((D,), jnp.float32),
            pltpu.VMEM((D,), jnp.float32),
            pltpu.SemaphoreType.DMA,
            pltpu.SemaphoreType.DMA,
        ],
        compiler_params=pltpu.CompilerParams(use_tc_tiling_on_sc=False),
    )(x3, table, wv, bv)


def kernel(x, table, W, b):
    xf = x.astype(jnp.int32).reshape(NCHUNK_G, IDX_PER_CHUNK)
    xp = jnp.pad(xf, ((0, 0), (0, IDX_PAD * GATHER_W - IDX_PER_CHUNK)))
    x3 = xp.reshape(NCHUNK_G, IDX_PAD, GATHER_W)
    wv = (W.reshape(D) / jnp.float32(L)).astype(jnp.float32)
    bv = jnp.broadcast_to(b.astype(jnp.float32), (D,))
    out = _run(x3, table, wv, bv)
    return out.reshape(B, 1)


# speedup vs baseline: 18.0444x; 1.8828x over previous
import jax
import jax.numpy as jnp
from jax import lax
from jax.experimental import pallas as pl
from jax.experimental.pallas import tpu as pltpu
from jax.experimental.pallas import tpu_sc as plsc

B = 16384
L = 200
D = 16
V = 1000000
NW = 32
ROWS_PER_W = B // NW
CHUNK = 16
NCHUNK = ROWS_PER_W // CHUNK
NCHUNK_G = B // CHUNK
GATHER_W = 128
TAIL_W = L - GATHER_W
ROWS_PER_ELEM = 2
BLK_ROWS = CHUNK * ROWS_PER_ELEM

PROJ_BLK = 4096


def _project_body(tt_ref, w_ref, t_ref):
    acc = jnp.zeros((PROJ_BLK,), jnp.float32)
    for k in range(D):
        acc = acc + tt_ref[k, :] * w_ref[0, k]
    t_ref[...] = acc


def _project(tt, wv):
    return pl.pallas_call(
        _project_body,
        out_shape=jax.ShapeDtypeStruct((V,), jnp.float32),
        grid=(V // PROJ_BLK,),
        in_specs=[
            pl.BlockSpec((D, PROJ_BLK), lambda i: (0, i)),
            pl.BlockSpec((1, D), lambda i: (0, 0)),
        ],
        out_specs=pl.BlockSpec((PROJ_BLK,), lambda i: (i,)),
    )(tt, wv)


def _body(x3, t, bv, out, idx_v, val_v, b_v, o16_v, gs0, gs1):
    wid = lax.axis_index("s") * 2 + lax.axis_index("c")

    pltpu.sync_copy(bv, b_v)

    lane = lax.iota(jnp.int32, D)

    def fire(c, p, sem):
        pltpu.sync_copy(x3.at[c], idx_v.at[p])

        def go(e, _):
            pltpu.async_copy(
                t.at[idx_v.at[p].at[2 * e]], val_v.at[p].at[2 * e], sem
            )
            pltpu.async_copy(
                t.at[idx_v.at[p, 2 * e + 1, pl.ds(0, TAIL_W)]],
                val_v.at[p, 2 * e + 1, pl.ds(0, TAIL_W)],
                sem,
            )
            return 0

        lax.fori_loop(0, CHUNK, go, 0)

    def drain(p, sem):
        def go(e, _):
            pltpu.make_async_copy(
                t.at[idx_v.at[p].at[2 * e]], val_v.at[p].at[2 * e], sem
            ).wait()
            pltpu.make_async_copy(
                t.at[idx_v.at[p, 2 * e + 1, pl.ds(0, TAIL_W)]],
                val_v.at[p, 2 * e + 1, pl.ds(0, TAIL_W)],
                sem,
            ).wait()
            return 0

        lax.fori_loop(0, CHUNK, go, 0)

    def compute(c, p):
        vp = val_v.at[p]
        tail_mask = lane < jnp.int32(TAIL_W - 4 * D)

        def elem_body(e, sel):
            r0 = 2 * e
            r1 = 2 * e + 1
            v0 = jnp.zeros((D,), jnp.float32)
            v1 = jnp.zeros((D,), jnp.float32)
            for k in range(GATHER_W // D):
                if k % 2 == 0:
                    v0 = v0 + vp[r0, pl.ds(k * D, D)]
                else:
                    v1 = v1 + vp[r0, pl.ds(k * D, D)]
            for k in range(TAIL_W // D):
                if k % 2 == 0:
                    v0 = v0 + vp[r1, pl.ds(k * D, D)]
                else:
                    v1 = v1 + vp[r1, pl.ds(k * D, D)]
            tail = vp[r1, pl.ds((TAIL_W // D) * D, D)]
            v = v0 + v1 + jnp.where(tail_mask, tail, 0.0)
            dnums = lax.GatherDimensionNumbers(
                offset_dims=(), collapsed_slice_dims=(0,), start_index_map=(0,)
            )
            for sh in (8, 4, 2, 1):
                v = v + lax.gather(
                    v,
                    (lane ^ sh)[:, None],
                    dnums,
                    (1,),
                    mode=lax.GatherScatterMode.PROMISE_IN_BOUNDS,
                )
            return jnp.where(lane == e, v, sel)

        sel = lax.fori_loop(0, CHUNK, elem_body, jnp.zeros((D,), jnp.float32))
        zval = sel + b_v[...]
        sig = 1.0 / (1.0 + jnp.exp(-zval))
        r = (sig * 10000.0 + 0.5).astype(jnp.int32).astype(jnp.float32) * 1e-4
        o16_v[...] = r
        pltpu.sync_copy(o16_v, out.at[pl.ds(c * CHUNK, CHUNK)])

    c0 = wid * NCHUNK
    fire(c0, 0, gs0)

    def pair(k, _):
        ca = c0 + 2 * k
        fire(ca + 1, 1, gs1)
        drain(0, gs0)
        compute(ca, 0)

        @pl.when(k < NCHUNK // 2 - 1)
        def _prefetch():
            fire(ca + 2, 0, gs0)

        drain(1, gs1)
        compute(ca + 1, 1)
        return 0

    lax.fori_loop(0, NCHUNK // 2, pair, 0)


@jax.jit
def _run(x3, table, W, b):
    tt = table.T
    wv = (W / jnp.float32(L)).astype(jnp.float32)
    t = _project(tt, wv)
    bv = jnp.broadcast_to(b.astype(jnp.float32), (D,))
    mesh = plsc.VectorSubcoreMesh(core_axis_name="c", subcore_axis_name="s")
    return pl.kernel(
        _body,
        out_type=jax.ShapeDtypeStruct((B,), jnp.float32),
        mesh=mesh,
        scratch_types=[
            pltpu.VMEM((2, BLK_ROWS, GATHER_W), jnp.int32),
            pltpu.VMEM((2, BLK_ROWS, GATHER_W), jnp.float32),
            pltpu.VMEM((D,), jnp.float32),
            pltpu.VMEM((D,), jnp.float32),
            pltpu.SemaphoreType.DMA,
            pltpu.SemaphoreType.DMA,
        ],
        compiler_params=pltpu.CompilerParams(use_tc_tiling_on_sc=False),
    )(x3, t, bv)


def kernel(x, table, W, b):
    xp = jnp.pad(x.astype(jnp.int32), ((0, 0), (0, ROWS_PER_ELEM * GATHER_W - L)))
    x3 = xp.reshape(NCHUNK_G, BLK_ROWS, GATHER_W)
    out = _run(x3, table, W, b)
    return out.reshape(B, 1)


# speedup vs baseline: 18.4547x; 1.0227x over previous
import jax
import jax.numpy as jnp
from jax import lax
from jax.experimental import pallas as pl
from jax.experimental.pallas import tpu as pltpu
from jax.experimental.pallas import tpu_sc as plsc

B = 16384
L = 200
D = 16
V = 1000000
NW = 32
ROWS_PER_W = B // NW
CHUNK = 16
NCHUNK = ROWS_PER_W // CHUNK
IDX_PER_CHUNK = CHUNK * L
GATHER_W = 128
NGATHER = IDX_PER_CHUNK // GATHER_W

PROJ_BLK = 4096


def _project_body(tt_ref, w_ref, t_ref):
    acc = jnp.zeros((PROJ_BLK,), jnp.float32)
    for k in range(D):
        acc = acc + tt_ref[k, :] * w_ref[0, k]
    t_ref[...] = acc


def _project(tt, wv):
    return pl.pallas_call(
        _project_body,
        out_shape=jax.ShapeDtypeStruct((V,), jnp.float32),
        grid=(V // PROJ_BLK,),
        in_specs=[
            pl.BlockSpec((D, PROJ_BLK), lambda i: (0, i)),
            pl.BlockSpec((1, D), lambda i: (0, 0)),
        ],
        out_specs=pl.BlockSpec((PROJ_BLK,), lambda i: (i,)),
    )(tt, wv)


def _body(xf, t, bv, out, idx0_v, idx1_v, val0_v, val1_v, b_v, o16_v, gs0, gs1):
    wid = lax.axis_index("s") * 2 + lax.axis_index("c")

    pltpu.sync_copy(bv, b_v)

    lane = lax.iota(jnp.int32, D)

    def fire(c, idx_v, val_v, sem):
        pltpu.sync_copy(xf.at[pl.ds(c * IDX_PER_CHUNK, IDX_PER_CHUNK)], idx_v)

        def go(e, _):
            pltpu.async_copy(
                t.at[idx_v.at[pl.ds(e * L, GATHER_W)]],
                val_v.at[e, pl.ds(0, GATHER_W)],
                sem,
            )
            pltpu.async_copy(
                t.at[idx_v.at[pl.ds(e * L + GATHER_W, L - GATHER_W)]],
                val_v.at[e, pl.ds(GATHER_W, L - GATHER_W)],
                sem,
            )
            return 0

        lax.fori_loop(0, CHUNK, go, 0)

    def drain(idx_v, val_v, sem):
        def go(e, _):
            pltpu.make_async_copy(
                t.at[idx_v.at[pl.ds(e * L, GATHER_W)]],
                val_v.at[e, pl.ds(0, GATHER_W)],
                sem,
            ).wait()
            pltpu.make_async_copy(
                t.at[idx_v.at[pl.ds(e * L + GATHER_W, L - GATHER_W)]],
                val_v.at[e, pl.ds(GATHER_W, L - GATHER_W)],
                sem,
            ).wait()
            return 0

        lax.fori_loop(0, CHUNK, go, 0)

    def compute(c, vp):

        tail_mask = lane < jnp.int32(L - (L // D) * D)

        def elem_body(e, sel):
            v0 = jnp.zeros((D,), jnp.float32)
            v1 = jnp.zeros((D,), jnp.float32)
            for k in range(L // D):
                g = vp[e, pl.ds(k * D, D)]
                if k % 2 == 0:
                    v0 = v0 + g
                else:
                    v1 = v1 + g
            tail = vp[e, pl.ds((L // D) * D, D)]
            v = v0 + v1 + jnp.where(tail_mask, tail, 0.0)
            dnums = lax.GatherDimensionNumbers(
                offset_dims=(), collapsed_slice_dims=(0,), start_index_map=(0,)
            )
            for sh in (8, 4, 2, 1):
                v = v + lax.gather(
                    v,
                    (lane ^ sh)[:, None],
                    dnums,
                    (1,),
                    mode=lax.GatherScatterMode.PROMISE_IN_BOUNDS,
                )
            return jnp.where(lane == e, v, sel)

        sel = lax.fori_loop(0, CHUNK, elem_body, jnp.zeros((D,), jnp.float32))
        zval = sel + b_v[...]
        sig = 1.0 / (1.0 + jnp.exp(-zval))
        r = (sig * 10000.0 + 0.5).astype(jnp.int32).astype(jnp.float32) * 1e-4
        o16_v[...] = r
        pltpu.sync_copy(o16_v, out.at[pl.ds(c * CHUNK, CHUNK)])

    c0 = wid * NCHUNK
    fire(c0, idx0_v, val0_v, gs0)

    def pair(k, _):
        ca = c0 + 2 * k
        fire(ca + 1, idx1_v, val1_v, gs1)
        drain(idx0_v, val0_v, gs0)
        compute(ca, val0_v)

        @pl.when(k < NCHUNK // 2 - 1)
        def _prefetch():
            fire(ca + 2, idx0_v, val0_v, gs0)

        drain(idx1_v, val1_v, gs1)
        compute(ca + 1, val1_v)
        return 0

    lax.fori_loop(0, NCHUNK // 2, pair, 0)


@jax.jit
def _run(xf, table, W, b):
    tt = table.T
    wv = (W / jnp.float32(L)).astype(jnp.float32)
    t = _project(tt, wv)
    bv = jnp.broadcast_to(b.astype(jnp.float32), (D,))
    mesh = plsc.VectorSubcoreMesh(core_axis_name="c", subcore_axis_name="s")
    return pl.kernel(
        _body,
        out_type=jax.ShapeDtypeStruct((B,), jnp.float32),
        mesh=mesh,
        scratch_types=[
            pltpu.VMEM((IDX_PER_CHUNK,), jnp.int32),
            pltpu.VMEM((IDX_PER_CHUNK,), jnp.int32),
            pltpu.VMEM((CHUNK, 2 * GATHER_W), jnp.float32),
            pltpu.VMEM((CHUNK, 2 * GATHER_W), jnp.float32),
            pltpu.VMEM((D,), jnp.float32),
            pltpu.VMEM((D,), jnp.float32),
            pltpu.SemaphoreType.DMA,
            pltpu.SemaphoreType.DMA,
        ],
        compiler_params=pltpu.CompilerParams(use_tc_tiling_on_sc=False),
    )(xf, t, bv)


def kernel(x, table, W, b):
    xf = x.astype(jnp.int32).reshape(B * L)
    out = _run(xf, table, W, b)
    return out.reshape(B, 1)
